# baseline (speedup 1.0000x reference)
import functools

import jax
import jax.numpy as jnp
from jax import lax
from jax.experimental import pallas as pl
from jax.experimental.pallas import tpu as pltpu
from jax.experimental.pallas import tpu_sc as plsc

B, N, M, K = 8, 8192, 1024, 32
FEAT = 29
IN_DIM = 32
HIDDEN = 64
OUT = 128

TM = 128


def _fps_body(xs_ref, ys_ref, zs_ref, cx_ref, cy_ref, cz_ref):
    xs = xs_ref[...]
    ys = ys_ref[...]
    zs = zs_ref[...]
    lane = lax.broadcasted_iota(jnp.int32, (1, N), 1)
    miota = lax.broadcasted_iota(jnp.int32, (1, M), 1)

    lx0 = xs[:, 0:1]
    ly0 = ys[:, 0:1]
    lz0 = zs[:, 0:1]
    cxs0 = jnp.where(miota == 0, lx0, jnp.zeros((B, M), jnp.float32))
    cys0 = jnp.where(miota == 0, ly0, jnp.zeros((B, M), jnp.float32))
    czs0 = jnp.where(miota == 0, lz0, jnp.zeros((B, M), jnp.float32))
    dists0 = jnp.full((B, N), jnp.inf, jnp.float32)

    def body(i, state):
        dists, lx, ly, lz, cxs, cys, czs = state
        dx = xs - lx
        dy = ys - ly
        dz = zs - lz
        d = (dx * dx + dy * dy) + dz * dz
        dists = jnp.minimum(dists, d)
        m = jnp.max(dists, axis=1, keepdims=True)
        sel = jnp.min(jnp.where(dists == m, lane, N), axis=1, keepdims=True)
        selm = lane == sel
        zero = jnp.zeros((B, N), jnp.float32)
        lx = jnp.sum(jnp.where(selm, xs, zero), axis=1, keepdims=True)
        ly = jnp.sum(jnp.where(selm, ys, zero), axis=1, keepdims=True)
        lz = jnp.sum(jnp.where(selm, zs, zero), axis=1, keepdims=True)
        hit = miota == i
        cxs = jnp.where(hit, lx, cxs)
        cys = jnp.where(hit, ly, cys)
        czs = jnp.where(hit, lz, czs)
        return (dists, lx, ly, lz, cxs, cys, czs)

    state = (dists0, lx0, ly0, lz0, cxs0, cys0, czs0)
    state = lax.fori_loop(1, M, body, state)
    _, _, _, _, cxs, cys, czs = state
    cx_ref[...] = cxs
    cy_ref[...] = cys
    cz_ref[...] = czs


def _fps(xs, ys, zs):
    out_shape = [jax.ShapeDtypeStruct((B, M), jnp.float32)] * 3
    return pl.pallas_call(_fps_body, out_shape=out_shape)(xs, ys, zs)


def _knn_body(xyzt_ref, cen_ref, idx_ref):
    b = pl.program_id(0)
    xs = xyzt_ref[0, 0:1, :]
    ys = xyzt_ref[0, 1:2, :]
    zs = xyzt_ref[0, 2:3, :]
    cen = cen_ref[0]
    cx = cen[:, 0:1]
    cy = cen[:, 1:2]
    cz = cen[:, 2:3]
    dx = cx - xs
    dy = cy - ys
    dz = cz - zs
    d2 = (dx * dx + dy * dy) + dz * dz

    kiota = lax.broadcasted_iota(jnp.int32, (1, K), 1)
    half = N // 2
    d2a = d2[:, :half]
    d2b = d2[:, half:]
    infh = jnp.full((TM, half), jnp.inf, jnp.float32)

    def body(k, state):
        lva, va, ia, lvb, vb, ib = state
        ma = jnp.where(d2a > lva, d2a, infh)
        mb = jnp.where(d2b > lvb, d2b, infh)
        lva = jnp.min(ma, axis=1, keepdims=True)
        lvb = jnp.min(mb, axis=1, keepdims=True)
        sa = jnp.argmin(ma, axis=1).reshape(TM, 1)
        sb = jnp.argmin(mb, axis=1).reshape(TM, 1)
        va = jnp.where(kiota == k, lva, va)
        ia = jnp.where(kiota == k, sa, ia)
        vb = jnp.where(kiota == (K - 1) - k, lvb, vb)
        ib = jnp.where(kiota == (K - 1) - k, sb + half, ib)
        return (lva, va, ia, lvb, vb, ib)

    z_f = jnp.zeros((TM, K), jnp.float32)
    z_i = jnp.zeros((TM, K), jnp.int32)
    lv0 = jnp.full((TM, 1), -jnp.inf, jnp.float32)
    _, va, ia, _, vb, ib = lax.fori_loop(
        0, K, body, (lv0, z_f, z_i, lv0, z_f, z_i))
    takea = va <= vb
    idx_ref[0] = jnp.where(takea, ia, ib) + b * N


def _knn(xyzt, centroids):
    grid = (B, M // TM)
    return pl.pallas_call(
        _knn_body,
        grid=grid,
        in_specs=[
            pl.BlockSpec((1, 3, N), lambda b, t: (b, 0, 0)),
            pl.BlockSpec((1, TM, 3), lambda b, t: (b, t, 0)),
        ],
        out_specs=pl.BlockSpec((1, TM, K), lambda b, t: (b, t, 0)),
        out_shape=jax.ShapeDtypeStruct((B, M, K), jnp.int32),
    )(xyzt, centroids)


def _sc_gather(table, idx):
    info = plsc.get_sparse_core_info()
    nc, ns = info.num_cores, info.num_subcores
    nw = nc * ns
    total = B * M * K
    per_w = total // nw
    ch = 2048
    nch = per_w // ch

    mesh = plsc.VectorSubcoreMesh(core_axis_name="c", subcore_axis_name="s")

    @functools.partial(
        pl.kernel,
        mesh=mesh,
        out_type=jax.ShapeDtypeStruct((total, IN_DIM), jnp.float32),
        compiler_params=pltpu.CompilerParams(use_tc_tiling_on_sc=False),
        scratch_types=[
            pltpu.VMEM((ch,), jnp.int32),
            pltpu.VMEM((ch, IN_DIM), jnp.float32),
            pltpu.SemaphoreType.DMA,
        ],
    )
    def gk(table_hbm, idx_hbm, out_hbm, idx_v, rows_v, sem):
        wid = lax.axis_index("s") * nc + lax.axis_index("c")
        for t in range(nch):
            base = wid * per_w + t * ch
            pltpu.sync_copy(idx_hbm.at[pl.ds(base, ch)], idx_v)
            pltpu.async_copy(table_hbm.at[idx_v], rows_v, sem).wait()
            pltpu.sync_copy(rows_v, out_hbm.at[pl.ds(base, ch)])

    return gk(table, idx)


def _mlp_body(g_ref, cen_ref, wh_ref, bh_ref, lg_ref, lb_ref, wo_ref,
              bo_ref, out_ref):
    g = g_ref[...]
    cen = cen_ref[...]
    cpad = jnp.concatenate(
        [cen, jnp.zeros((TM, IN_DIM - 3), jnp.float32)], axis=1)
    crep = jnp.broadcast_to(
        cpad.reshape(TM, 1, IN_DIM), (TM, K, IN_DIM)).reshape(TM * K, IN_DIM)
    x = g - crep
    h = lax.dot_general(x, wh_ref[...], (((1,), (1,)), ((), ())),
                        preferred_element_type=jnp.float32)
    h = h + bh_ref[...]
    mu = jnp.mean(h, axis=1, keepdims=True)
    hc = h - mu
    var = jnp.mean(hc * hc, axis=1, keepdims=True)
    h = hc / jnp.sqrt(var + 1e-5) * lg_ref[...] + lb_ref[...]
    y = lax.dot_general(h, wo_ref[...], (((1,), (1,)), ((), ())),
                        preferred_element_type=jnp.float32)
    y = y + bo_ref[...]
    y = y.reshape(TM, K, OUT)
    out_ref[...] = jnp.max(y, axis=1)


def _mlp(gathered, cen2d, W_h, b_h, ln_g, ln_b, W_o, b_o):
    grid = (B * M // TM,)
    return pl.pallas_call(
        _mlp_body,
        grid=grid,
        in_specs=[
            pl.BlockSpec((TM * K, IN_DIM), lambda t: (t, 0)),
            pl.BlockSpec((TM, 3), lambda t: (t, 0)),
            pl.BlockSpec((HIDDEN, IN_DIM), lambda t: (0, 0)),
            pl.BlockSpec((1, HIDDEN), lambda t: (0, 0)),
            pl.BlockSpec((1, HIDDEN), lambda t: (0, 0)),
            pl.BlockSpec((1, HIDDEN), lambda t: (0, 0)),
            pl.BlockSpec((OUT, HIDDEN), lambda t: (0, 0)),
            pl.BlockSpec((1, OUT), lambda t: (0, 0)),
        ],
        out_specs=pl.BlockSpec((TM, OUT), lambda t: (t, 0)),
        out_shape=jax.ShapeDtypeStruct((B * M, OUT), jnp.float32),
    )(gathered, cen2d, W_h, b_h.reshape(1, HIDDEN), ln_g.reshape(1, HIDDEN),
      ln_b.reshape(1, HIDDEN), W_o, b_o.reshape(1, OUT))


@jax.jit
def kernel(xyz, features, W_h, b_h, ln_g, ln_b, W_o, b_o):
    xyzt = jnp.transpose(xyz, (0, 2, 1))
    cx, cy, cz = _fps(xyzt[:, 0, :], xyzt[:, 1, :], xyzt[:, 2, :])
    centroids = jnp.stack([cx, cy, cz], axis=-1)
    idx = _knn(xyzt, centroids)
    table = jnp.concatenate([xyz, features], axis=-1).reshape(B * N, IN_DIM)
    gathered = _sc_gather(table, idx.reshape(B * M * K))
    out = _mlp(gathered, centroids.reshape(B * M, 3),
               W_h, b_h, ln_g, ln_b, W_o, b_o)
    return (centroids, out.reshape(B, M, OUT))


# speedup vs baseline: 1.2332x; 1.2332x over previous
import functools

import jax
import jax.numpy as jnp
from jax import lax
from jax.experimental import pallas as pl
from jax.experimental.pallas import tpu as pltpu
from jax.experimental.pallas import tpu_sc as plsc

B, N, M, K = 8, 8192, 1024, 32
FEAT = 29
IN_DIM = 32
HIDDEN = 64
OUT = 128

TM = 256
TMM = 128


def _fps_body(xs_ref, ys_ref, zs_ref, cx_ref, cy_ref, cz_ref):
    xs = xs_ref[...]
    ys = ys_ref[...]
    zs = zs_ref[...]
    lane = lax.broadcasted_iota(jnp.int32, (1, N), 1)
    miota = lax.broadcasted_iota(jnp.int32, (1, M), 1)

    lx0 = xs[:, 0:1]
    ly0 = ys[:, 0:1]
    lz0 = zs[:, 0:1]
    cxs0 = jnp.where(miota == 0, lx0, jnp.zeros((B, M), jnp.float32))
    cys0 = jnp.where(miota == 0, ly0, jnp.zeros((B, M), jnp.float32))
    czs0 = jnp.where(miota == 0, lz0, jnp.zeros((B, M), jnp.float32))
    dists0 = jnp.full((B, N), jnp.inf, jnp.float32)

    def body(i, state):
        dists, lx, ly, lz, cxs, cys, czs = state
        dx = xs - lx
        dy = ys - ly
        dz = zs - lz
        d = (dx * dx + dy * dy) + dz * dz
        dists = jnp.minimum(dists, d)
        m = jnp.max(dists, axis=1, keepdims=True)
        sel = jnp.min(jnp.where(dists == m, lane, N), axis=1, keepdims=True)
        selm = lane == sel
        zero = jnp.zeros((B, N), jnp.float32)
        lx = jnp.sum(jnp.where(selm, xs, zero), axis=1, keepdims=True)
        ly = jnp.sum(jnp.where(selm, ys, zero), axis=1, keepdims=True)
        lz = jnp.sum(jnp.where(selm, zs, zero), axis=1, keepdims=True)
        hit = miota == i
        cxs = jnp.where(hit, lx, cxs)
        cys = jnp.where(hit, ly, cys)
        czs = jnp.where(hit, lz, czs)
        return (dists, lx, ly, lz, cxs, cys, czs)

    state = (dists0, lx0, ly0, lz0, cxs0, cys0, czs0)
    state = lax.fori_loop(1, M, body, state)
    _, _, _, _, cxs, cys, czs = state
    cx_ref[...] = cxs
    cy_ref[...] = cys
    cz_ref[...] = czs


def _fps(xs, ys, zs):
    out_shape = [jax.ShapeDtypeStruct((B, M), jnp.float32)] * 3
    return pl.pallas_call(_fps_body, out_shape=out_shape)(xs, ys, zs)


def _knn_body(xyzt_ref, cen_ref, idx_ref):
    b = pl.program_id(0)
    xs = xyzt_ref[0, 0:1, :]
    ys = xyzt_ref[0, 1:2, :]
    zs = xyzt_ref[0, 2:3, :]
    cen = cen_ref[0]
    cx = cen[:, 0:1]
    cy = cen[:, 1:2]
    cz = cen[:, 2:3]
    dx = cx - xs
    dy = cy - ys
    dz = cz - zs
    d2 = (dx * dx + dy * dy) + dz * dz

    lane = lax.broadcasted_iota(jnp.int32, (1, N), 1)
    kiota = lax.broadcasted_iota(jnp.int32, (1, K), 1)
    inf = jnp.full((TM, N), jnp.inf, jnp.float32)

    def body(k, state):
        lv, acc = state
        masked = jnp.where(d2 > lv, d2, inf)
        lv = jnp.min(masked, axis=1, keepdims=True)
        sel = jnp.argmin(masked, axis=1).reshape(TM, 1)
        acc = jnp.where(kiota == k, sel, acc)
        return (lv, acc)

    acc0 = jnp.zeros((TM, K), jnp.int32)
    lv0 = jnp.full((TM, 1), -jnp.inf, jnp.float32)
    _, acc = lax.fori_loop(0, K, body, (lv0, acc0))
    idx_ref[0] = acc + b * N


def _knn(xyzt, centroids):
    grid = (B, M // TM)
    return pl.pallas_call(
        _knn_body,
        grid=grid,
        in_specs=[
            pl.BlockSpec((1, 3, N), lambda b, t: (b, 0, 0)),
            pl.BlockSpec((1, TM, 3), lambda b, t: (b, t, 0)),
        ],
        out_specs=pl.BlockSpec((1, TM, K), lambda b, t: (b, t, 0)),
        out_shape=jax.ShapeDtypeStruct((B, M, K), jnp.int32),
    )(xyzt, centroids)


def _sc_gather(table, idx):
    info = plsc.get_sparse_core_info()
    nc, ns = info.num_cores, info.num_subcores
    nw = nc * ns
    total = B * M * K
    per_w = total // nw
    ch = 2048
    nch = per_w // ch

    mesh = plsc.VectorSubcoreMesh(core_axis_name="c", subcore_axis_name="s")

    @functools.partial(
        pl.kernel,
        mesh=mesh,
        out_type=jax.ShapeDtypeStruct((total, IN_DIM), jnp.float32),
        compiler_params=pltpu.CompilerParams(use_tc_tiling_on_sc=False),
        scratch_types=[
            pltpu.VMEM((ch,), jnp.int32),
            pltpu.VMEM((ch, IN_DIM), jnp.float32),
            pltpu.SemaphoreType.DMA,
        ],
    )
    def gk(table_hbm, idx_hbm, out_hbm, idx_v, rows_v, sem):
        wid = lax.axis_index("s") * nc + lax.axis_index("c")
        for t in range(nch):
            base = wid * per_w + t * ch
            pltpu.sync_copy(idx_hbm.at[pl.ds(base, ch)], idx_v)
            pltpu.async_copy(table_hbm.at[idx_v], rows_v, sem).wait()
            pltpu.sync_copy(rows_v, out_hbm.at[pl.ds(base, ch)])

    return gk(table, idx)


def _mlp_body(g_ref, cen_ref, wh_ref, bh_ref, lg_ref, lb_ref, wo_ref,
              bo_ref, out_ref):
    g = g_ref[...]
    cen = cen_ref[...]
    cpad = jnp.concatenate(
        [cen, jnp.zeros((TMM, IN_DIM - 3), jnp.float32)], axis=1)
    crep = jnp.broadcast_to(
        cpad.reshape(TMM, 1, IN_DIM), (TMM, K, IN_DIM)).reshape(TMM * K, IN_DIM)
    x = g - crep
    h = lax.dot_general(x, wh_ref[...], (((1,), (1,)), ((), ())),
                        preferred_element_type=jnp.float32)
    h = h + bh_ref[...]
    mu = jnp.mean(h, axis=1, keepdims=True)
    hc = h - mu
    var = jnp.mean(hc * hc, axis=1, keepdims=True)
    h = hc / jnp.sqrt(var + 1e-5) * lg_ref[...] + lb_ref[...]
    y = lax.dot_general(h, wo_ref[...], (((1,), (1,)), ((), ())),
                        preferred_element_type=jnp.float32)
    y = y + bo_ref[...]
    y = y.reshape(TMM, K, OUT)
    out_ref[...] = jnp.max(y, axis=1)


def _mlp(gathered, cen2d, W_h, b_h, ln_g, ln_b, W_o, b_o):
    grid = (B * M // TMM,)
    return pl.pallas_call(
        _mlp_body,
        grid=grid,
        in_specs=[
            pl.BlockSpec((TMM * K, IN_DIM), lambda t: (t, 0)),
            pl.BlockSpec((TMM, 3), lambda t: (t, 0)),
            pl.BlockSpec((HIDDEN, IN_DIM), lambda t: (0, 0)),
            pl.BlockSpec((1, HIDDEN), lambda t: (0, 0)),
            pl.BlockSpec((1, HIDDEN), lambda t: (0, 0)),
            pl.BlockSpec((1, HIDDEN), lambda t: (0, 0)),
            pl.BlockSpec((OUT, HIDDEN), lambda t: (0, 0)),
            pl.BlockSpec((1, OUT), lambda t: (0, 0)),
        ],
        out_specs=pl.BlockSpec((TMM, OUT), lambda t: (t, 0)),
        out_shape=jax.ShapeDtypeStruct((B * M, OUT), jnp.float32),
    )(gathered, cen2d, W_h, b_h.reshape(1, HIDDEN), ln_g.reshape(1, HIDDEN),
      ln_b.reshape(1, HIDDEN), W_o, b_o.reshape(1, OUT))


@jax.jit
def kernel(xyz, features, W_h, b_h, ln_g, ln_b, W_o, b_o):
    xyzt = jnp.transpose(xyz, (0, 2, 1))
    cx, cy, cz = _fps(xyzt[:, 0, :], xyzt[:, 1, :], xyzt[:, 2, :])
    centroids = jnp.stack([cx, cy, cz], axis=-1)
    idx = _knn(xyzt, centroids)
    table = jnp.concatenate([xyz, features], axis=-1).reshape(B * N, IN_DIM)
    gathered = _sc_gather(table, idx.reshape(B * M * K))
    out = _mlp(gathered, centroids.reshape(B * M, 3),
               W_h, b_h, ln_g, ln_b, W_o, b_o)
    return (centroids, out.reshape(B, M, OUT))


# speedup vs baseline: 1.2498x; 1.0134x over previous
import functools

import jax
import jax.numpy as jnp
from jax import lax
from jax.experimental import pallas as pl
from jax.experimental.pallas import tpu as pltpu
from jax.experimental.pallas import tpu_sc as plsc

B, N, M, K = 8, 8192, 1024, 32
FEAT = 29
IN_DIM = 32
HIDDEN = 64
OUT = 128

TM = 512
TMM = 128


def _fps_body(xs_ref, ys_ref, zs_ref, cx_ref, cy_ref, cz_ref):
    xs = xs_ref[...]
    ys = ys_ref[...]
    zs = zs_ref[...]
    lane = lax.broadcasted_iota(jnp.int32, (1, N), 1)
    miota = lax.broadcasted_iota(jnp.int32, (1, M), 1)

    lx0 = xs[:, 0:1]
    ly0 = ys[:, 0:1]
    lz0 = zs[:, 0:1]
    cxs0 = jnp.where(miota == 0, lx0, jnp.zeros((B, M), jnp.float32))
    cys0 = jnp.where(miota == 0, ly0, jnp.zeros((B, M), jnp.float32))
    czs0 = jnp.where(miota == 0, lz0, jnp.zeros((B, M), jnp.float32))
    dists0 = jnp.full((B, N), jnp.inf, jnp.float32)

    def body(i, state):
        dists, lx, ly, lz, cxs, cys, czs = state
        dx = xs - lx
        dy = ys - ly
        dz = zs - lz
        d = (dx * dx + dy * dy) + dz * dz
        dists = jnp.minimum(dists, d)
        m = jnp.max(dists, axis=1, keepdims=True)
        sel = jnp.min(jnp.where(dists == m, lane, N), axis=1, keepdims=True)
        selm = lane == sel
        zero = jnp.zeros((B, N), jnp.float32)
        lx = jnp.sum(jnp.where(selm, xs, zero), axis=1, keepdims=True)
        ly = jnp.sum(jnp.where(selm, ys, zero), axis=1, keepdims=True)
        lz = jnp.sum(jnp.where(selm, zs, zero), axis=1, keepdims=True)
        hit = miota == i
        cxs = jnp.where(hit, lx, cxs)
        cys = jnp.where(hit, ly, cys)
        czs = jnp.where(hit, lz, czs)
        return (dists, lx, ly, lz, cxs, cys, czs)

    state = (dists0, lx0, ly0, lz0, cxs0, cys0, czs0)
    state = lax.fori_loop(1, M, body, state)
    _, _, _, _, cxs, cys, czs = state
    cx_ref[...] = cxs
    cy_ref[...] = cys
    cz_ref[...] = czs


def _fps(xs, ys, zs):
    out_shape = [jax.ShapeDtypeStruct((B, M), jnp.float32)] * 3
    return pl.pallas_call(_fps_body, out_shape=out_shape)(xs, ys, zs)


def _knn_body(xyzt_ref, cen_ref, idx_ref):
    b = pl.program_id(0)
    xs = xyzt_ref[0, 0:1, :]
    ys = xyzt_ref[0, 1:2, :]
    zs = xyzt_ref[0, 2:3, :]
    cen = cen_ref[0]
    cx = cen[:, 0:1]
    cy = cen[:, 1:2]
    cz = cen[:, 2:3]
    dx = cx - xs
    dy = cy - ys
    dz = cz - zs
    d2 = (dx * dx + dy * dy) + dz * dz

    lane = lax.broadcasted_iota(jnp.int32, (1, N), 1)
    kiota = lax.broadcasted_iota(jnp.int32, (1, K), 1)
    inf = jnp.full((TM, N), jnp.inf, jnp.float32)

    def body(k, state):
        lv, acc = state
        masked = jnp.where(d2 > lv, d2, inf)
        lv = jnp.min(masked, axis=1, keepdims=True)
        sel = jnp.argmin(masked, axis=1).reshape(TM, 1)
        acc = jnp.where(kiota == k, sel, acc)
        return (lv, acc)

    acc0 = jnp.zeros((TM, K), jnp.int32)
    lv0 = jnp.full((TM, 1), -jnp.inf, jnp.float32)
    _, acc = lax.fori_loop(0, K, body, (lv0, acc0))
    idx_ref[0] = acc + b * N


def _knn(xyzt, centroids):
    grid = (B, M // TM)
    return pl.pallas_call(
        _knn_body,
        grid=grid,
        in_specs=[
            pl.BlockSpec((1, 3, N), lambda b, t: (b, 0, 0)),
            pl.BlockSpec((1, TM, 3), lambda b, t: (b, t, 0)),
        ],
        out_specs=pl.BlockSpec((1, TM, K), lambda b, t: (b, t, 0)),
        out_shape=jax.ShapeDtypeStruct((B, M, K), jnp.int32),
    )(xyzt, centroids)


def _sc_gather(table, idx):
    info = plsc.get_sparse_core_info()
    nc, ns = info.num_cores, info.num_subcores
    nw = nc * ns
    total = B * M * K
    per_w = total // nw
    ch = 2048
    nch = per_w // ch

    mesh = plsc.VectorSubcoreMesh(core_axis_name="c", subcore_axis_name="s")

    @functools.partial(
        pl.kernel,
        mesh=mesh,
        out_type=jax.ShapeDtypeStruct((total, IN_DIM), jnp.float32),
        compiler_params=pltpu.CompilerParams(use_tc_tiling_on_sc=False),
        scratch_types=[
            pltpu.VMEM((ch,), jnp.int32),
            pltpu.VMEM((ch, IN_DIM), jnp.float32),
            pltpu.SemaphoreType.DMA,
        ],
    )
    def gk(table_hbm, idx_hbm, out_hbm, idx_v, rows_v, sem):
        wid = lax.axis_index("s") * nc + lax.axis_index("c")
        for t in range(nch):
            base = wid * per_w + t * ch
            pltpu.sync_copy(idx_hbm.at[pl.ds(base, ch)], idx_v)
            pltpu.async_copy(table_hbm.at[idx_v], rows_v, sem).wait()
            pltpu.sync_copy(rows_v, out_hbm.at[pl.ds(base, ch)])

    return gk(table, idx)


def _mlp_body(g_ref, cen_ref, wh_ref, bh_ref, lg_ref, lb_ref, wo_ref,
              bo_ref, out_ref):
    g = g_ref[...]
    cen = cen_ref[...]
    cpad = jnp.concatenate(
        [cen, jnp.zeros((TMM, IN_DIM - 3), jnp.float32)], axis=1)
    crep = jnp.broadcast_to(
        cpad.reshape(TMM, 1, IN_DIM), (TMM, K, IN_DIM)).reshape(TMM * K, IN_DIM)
    x = g - crep
    h = lax.dot_general(x, wh_ref[...], (((1,), (1,)), ((), ())),
                        preferred_element_type=jnp.float32)
    h = h + bh_ref[...]
    mu = jnp.mean(h, axis=1, keepdims=True)
    hc = h - mu
    var = jnp.mean(hc * hc, axis=1, keepdims=True)
    h = hc / jnp.sqrt(var + 1e-5) * lg_ref[...] + lb_ref[...]
    y = lax.dot_general(h, wo_ref[...], (((1,), (1,)), ((), ())),
                        preferred_element_type=jnp.float32)
    y = y + bo_ref[...]
    y = y.reshape(TMM, K, OUT)
    out_ref[...] = jnp.max(y, axis=1)


def _mlp(gathered, cen2d, W_h, b_h, ln_g, ln_b, W_o, b_o):
    grid = (B * M // TMM,)
    return pl.pallas_call(
        _mlp_body,
        grid=grid,
        in_specs=[
            pl.BlockSpec((TMM * K, IN_DIM), lambda t: (t, 0)),
            pl.BlockSpec((TMM, 3), lambda t: (t, 0)),
            pl.BlockSpec((HIDDEN, IN_DIM), lambda t: (0, 0)),
            pl.BlockSpec((1, HIDDEN), lambda t: (0, 0)),
            pl.BlockSpec((1, HIDDEN), lambda t: (0, 0)),
            pl.BlockSpec((1, HIDDEN), lambda t: (0, 0)),
            pl.BlockSpec((OUT, HIDDEN), lambda t: (0, 0)),
            pl.BlockSpec((1, OUT), lambda t: (0, 0)),
        ],
        out_specs=pl.BlockSpec((TMM, OUT), lambda t: (t, 0)),
        out_shape=jax.ShapeDtypeStruct((B * M, OUT), jnp.float32),
    )(gathered, cen2d, W_h, b_h.reshape(1, HIDDEN), ln_g.reshape(1, HIDDEN),
      ln_b.reshape(1, HIDDEN), W_o, b_o.reshape(1, OUT))


@jax.jit
def kernel(xyz, features, W_h, b_h, ln_g, ln_b, W_o, b_o):
    xyzt = jnp.transpose(xyz, (0, 2, 1))
    cx, cy, cz = _fps(xyzt[:, 0, :], xyzt[:, 1, :], xyzt[:, 2, :])
    centroids = jnp.stack([cx, cy, cz], axis=-1)
    idx = _knn(xyzt, centroids)
    table = jnp.concatenate([xyz, features], axis=-1).reshape(B * N, IN_DIM)
    gathered = _sc_gather(table, idx.reshape(B * M * K))
    out = _mlp(gathered, centroids.reshape(B * M, 3),
               W_h, b_h, ln_g, ln_b, W_o, b_o)
    return (centroids, out.reshape(B, M, OUT))
